# baseline (speedup 1.0000x reference)
import functools

import jax
import jax.numpy as jnp
from jax import lax
from jax.experimental import pallas as pl
from jax.experimental.pallas import tpu as pltpu
from jax.experimental.pallas import tpu_sc as plsc

_NUM_CORES = 2
_NUM_SUBCORES = 16
_BLK_EDGES = 128


def _round_up(a, m):
    return (a + m - 1) // m * m


def _dense_two(h, W_rel, W_root, b, blk_rows):
    R, D = h.shape

    def body(h_ref, wr_ref, wo_ref, b_ref, m_ref, r_ref):
        hb = h_ref[...]
        dn = (((1,), (1,)), ((), ()))
        m_ref[...] = lax.dot_general(hb, wr_ref[...], dn,
                                     preferred_element_type=jnp.float32)
        r_ref[...] = lax.dot_general(hb, wo_ref[...], dn,
                                     preferred_element_type=jnp.float32) + b_ref[...]

    return pl.pallas_call(
        body,
        grid=(R // blk_rows,),
        in_specs=[
            pl.BlockSpec((blk_rows, D), lambda i: (i, 0)),
            pl.BlockSpec((D, D), lambda i: (0, 0)),
            pl.BlockSpec((D, D), lambda i: (0, 0)),
            pl.BlockSpec((1, D), lambda i: (0, 0)),
        ],
        out_specs=[
            pl.BlockSpec((blk_rows, D), lambda i: (i, 0)),
            pl.BlockSpec((blk_rows, D), lambda i: (i, 0)),
        ],
        out_shape=[
            jax.ShapeDtypeStruct((R, D), jnp.float32),
            jax.ShapeDtypeStruct((R, D), jnp.float32),
        ],
    )(h, W_rel, W_root, b)


def _fused_dense_two(parts, r_prev, W_rel, W_root, b, blk_rows):
    _, R, D = parts.shape

    def body(p_ref, rp_ref, wr_ref, wo_ref, b_ref, m_ref, r_ref):
        hb = jnp.maximum(p_ref[0] + p_ref[1] + rp_ref[...], 0.0)
        dn = (((1,), (1,)), ((), ()))
        m_ref[...] = lax.dot_general(hb, wr_ref[...], dn,
                                     preferred_element_type=jnp.float32)
        r_ref[...] = lax.dot_general(hb, wo_ref[...], dn,
                                     preferred_element_type=jnp.float32) + b_ref[...]

    return pl.pallas_call(
        body,
        grid=(R // blk_rows,),
        in_specs=[
            pl.BlockSpec((2, blk_rows, D), lambda i: (0, i, 0)),
            pl.BlockSpec((blk_rows, D), lambda i: (i, 0)),
            pl.BlockSpec((D, D), lambda i: (0, 0)),
            pl.BlockSpec((D, D), lambda i: (0, 0)),
            pl.BlockSpec((1, D), lambda i: (0, 0)),
        ],
        out_specs=[
            pl.BlockSpec((blk_rows, D), lambda i: (i, 0)),
            pl.BlockSpec((blk_rows, D), lambda i: (i, 0)),
        ],
        out_shape=[
            jax.ShapeDtypeStruct((R, D), jnp.float32),
            jax.ShapeDtypeStruct((R, D), jnp.float32),
        ],
    )(parts, r_prev, W_rel, W_root, b)


def _combine(parts, r, blk_rows):
    _, R, D = parts.shape

    def body(p_ref, r_ref, o_ref):
        o_ref[...] = jnp.maximum(p_ref[0] + p_ref[1] + r_ref[...], 0.0)

    return pl.pallas_call(
        body,
        grid=(R // blk_rows,),
        in_specs=[
            pl.BlockSpec((2, blk_rows, D), lambda i: (0, i, 0)),
            pl.BlockSpec((blk_rows, D), lambda i: (i, 0)),
        ],
        out_specs=pl.BlockSpec((blk_rows, D), lambda i: (i, 0)),
        out_shape=jax.ShapeDtypeStruct((R, D), jnp.float32),
    )(parts, r)


def _sc_segsum(m, sdb, zeros, n_acc, rows_per_tile, blocks_per_tile):
    D = m.shape[1]
    bpt = blocks_per_tile
    mesh = plsc.VectorSubcoreMesh(core_axis_name="c", subcore_axis_name="s",
                                  num_cores=_NUM_CORES,
                                  num_subcores=_NUM_SUBCORES)

    nb = 3
    assert bpt % nb == 0

    @functools.partial(
        pl.kernel,
        out_type=jax.ShapeDtypeStruct((_NUM_CORES, n_acc, D), jnp.float32),
        mesh=mesh,
        scratch_types=[
            pltpu.VMEM((nb, 2, _BLK_EDGES), jnp.int32),
            pltpu.VMEM((nb, _BLK_EDGES, D), jnp.float32),
            pltpu.VMEM_SHARED((n_acc, D), jnp.float32),
        ] + [pltpu.SemaphoreType.DMA] * nb,
    )
    def k(m_hbm, sdb_hbm, z_hbm, out_hbm, idx, rows, acc_sh, *gsem):
        c = lax.axis_index("c")
        s = lax.axis_index("s")
        my_rows = pl.ds(s * rows_per_tile, rows_per_tile)
        base = (c * _NUM_SUBCORES + s) * bpt
        pltpu.sync_copy(z_hbm.at[my_rows], acc_sh.at[my_rows])
        plsc.subcore_barrier()

        @pl.loop(0, bpt // nb)
        def _(i):
            pltpu.sync_copy(sdb_hbm.at[pl.ds(base + i * nb, nb)], idx)
            descs = [
                pltpu.async_copy(m_hbm.at[idx.at[b].at[0]], rows.at[b],
                                 gsem[b])
                for b in range(nb)
            ]
            for b in range(nb):
                descs[b].wait()
                pltpu.sync_copy(rows.at[b], acc_sh.at[idx.at[b].at[1]],
                                add=True)

        plsc.subcore_barrier()
        pltpu.sync_copy(acc_sh.at[my_rows], out_hbm.at[c].at[my_rows])

    return k(m, sdb, zeros)


def kernel(x, edge_index, W1_rel, W1_root, b1, W2_rel, W2_root, b2):
    N, D = x.shape
    E = edge_index.shape[1]
    nw = _NUM_CORES * _NUM_SUBCORES

    blocks_per_tile = _round_up(-(-E // (nw * _BLK_EDGES)), 3)
    e_pad = nw * _BLK_EDGES * blocks_per_tile
    rows_per_tile = _round_up(-(-(N + 1) // _NUM_SUBCORES), 8)
    n_acc = _NUM_SUBCORES * rows_per_tile

    src = edge_index[0].astype(jnp.int32)
    dst = edge_index[1].astype(jnp.int32)
    pad = e_pad - E
    nblocks = e_pad // _BLK_EDGES
    srcb = jnp.pad(src, (0, pad), constant_values=N).reshape(nblocks,
                                                             _BLK_EDGES)
    dstb = jnp.pad(dst, (0, pad), constant_values=N).reshape(nblocks,
                                                             _BLK_EDGES)
    sdb = jnp.stack([srcb, dstb], axis=1)
    xp = jnp.pad(x, ((0, n_acc - N), (0, 0)))
    zeros = jnp.zeros((n_acc, D), jnp.float32)
    b1r = b1.reshape(1, D)
    b2r = b2.reshape(1, D)

    blk_rows = 1024 if n_acc % 1024 == 0 else 64

    m1, r1 = _dense_two(xp, W1_rel, W1_root, b1r, blk_rows)
    parts1 = _sc_segsum(m1, sdb, zeros, n_acc, rows_per_tile, blocks_per_tile)
    m2, r2 = _fused_dense_two(parts1, r1, W2_rel, W2_root, b2r, blk_rows)
    parts2 = _sc_segsum(m2, sdb, zeros, n_acc, rows_per_tile, blocks_per_tile)
    out = _combine(parts2, r2, blk_rows)
    return out[:N]


# speedup vs baseline: 1.1927x; 1.1927x over previous
import functools

import jax
import jax.numpy as jnp
from jax import lax
from jax.experimental import pallas as pl
from jax.experimental.pallas import tpu as pltpu
from jax.experimental.pallas import tpu_sc as plsc

_NUM_CORES = 2
_NUM_SUBCORES = 16
_BLK_EDGES = 128


def _round_up(a, m):
    return (a + m - 1) // m * m


def _dense_two(h, W_rel, W_root, b, blk_rows):
    R, D = h.shape

    def body(h_ref, wr_ref, wo_ref, b_ref, m_ref, r_ref):
        hb = h_ref[...]
        dn = (((1,), (1,)), ((), ()))
        m_ref[...] = lax.dot_general(hb, wr_ref[...], dn,
                                     preferred_element_type=jnp.float32)
        r_ref[...] = lax.dot_general(hb, wo_ref[...], dn,
                                     preferred_element_type=jnp.float32) + b_ref[...]

    return pl.pallas_call(
        body,
        grid=(R // blk_rows,),
        in_specs=[
            pl.BlockSpec((blk_rows, D), lambda i: (i, 0)),
            pl.BlockSpec((D, D), lambda i: (0, 0)),
            pl.BlockSpec((D, D), lambda i: (0, 0)),
            pl.BlockSpec((1, D), lambda i: (0, 0)),
        ],
        out_specs=[
            pl.BlockSpec((blk_rows, D), lambda i: (i, 0)),
            pl.BlockSpec((blk_rows, D), lambda i: (i, 0)),
        ],
        out_shape=[
            jax.ShapeDtypeStruct((R, D), jnp.float32),
            jax.ShapeDtypeStruct((R, D), jnp.float32),
        ],
    )(h, W_rel, W_root, b)


def _fused_dense_two(parts, r_prev, W_rel, W_root, b, blk_rows):
    _, R, D = parts.shape

    def body(p_ref, rp_ref, wr_ref, wo_ref, b_ref, m_ref, r_ref):
        hb = jnp.maximum(p_ref[0] + p_ref[1] + rp_ref[...], 0.0)
        dn = (((1,), (1,)), ((), ()))
        m_ref[...] = lax.dot_general(hb, wr_ref[...], dn,
                                     preferred_element_type=jnp.float32)
        r_ref[...] = lax.dot_general(hb, wo_ref[...], dn,
                                     preferred_element_type=jnp.float32) + b_ref[...]

    return pl.pallas_call(
        body,
        grid=(R // blk_rows,),
        in_specs=[
            pl.BlockSpec((2, blk_rows, D), lambda i: (0, i, 0)),
            pl.BlockSpec((blk_rows, D), lambda i: (i, 0)),
            pl.BlockSpec((D, D), lambda i: (0, 0)),
            pl.BlockSpec((D, D), lambda i: (0, 0)),
            pl.BlockSpec((1, D), lambda i: (0, 0)),
        ],
        out_specs=[
            pl.BlockSpec((blk_rows, D), lambda i: (i, 0)),
            pl.BlockSpec((blk_rows, D), lambda i: (i, 0)),
        ],
        out_shape=[
            jax.ShapeDtypeStruct((R, D), jnp.float32),
            jax.ShapeDtypeStruct((R, D), jnp.float32),
        ],
    )(parts, r_prev, W_rel, W_root, b)


def _combine(parts, r, blk_rows):
    _, R, D = parts.shape

    def body(p_ref, r_ref, o_ref):
        o_ref[...] = jnp.maximum(p_ref[0] + p_ref[1] + r_ref[...], 0.0)

    return pl.pallas_call(
        body,
        grid=(R // blk_rows,),
        in_specs=[
            pl.BlockSpec((2, blk_rows, D), lambda i: (0, i, 0)),
            pl.BlockSpec((blk_rows, D), lambda i: (i, 0)),
        ],
        out_specs=pl.BlockSpec((blk_rows, D), lambda i: (i, 0)),
        out_shape=jax.ShapeDtypeStruct((R, D), jnp.float32),
    )(parts, r)


def _sc_segsum(m, sdb, zeros, n_acc, rows_per_tile, bpt0, bpt1):
    D = m.shape[1]
    mesh = plsc.VectorSubcoreMesh(core_axis_name="c", subcore_axis_name="s",
                                  num_cores=_NUM_CORES,
                                  num_subcores=_NUM_SUBCORES)

    nb = 3
    assert bpt0 % nb == 0 and bpt1 % nb == 0

    @functools.partial(
        pl.kernel,
        out_type=jax.ShapeDtypeStruct((_NUM_CORES, n_acc, D), jnp.float32),
        mesh=mesh,
        scratch_types=[
            pltpu.VMEM((nb, 2, _BLK_EDGES), jnp.int32),
            pltpu.VMEM((nb, _BLK_EDGES, D), jnp.float32),
            pltpu.VMEM_SHARED((n_acc, D), jnp.float32),
        ] + [pltpu.SemaphoreType.DMA] * nb,
    )
    def k(m_hbm, sdb_hbm, z_hbm, out_hbm, idx, rows, acc_sh, *gsem):
        c = lax.axis_index("c")
        s = lax.axis_index("s")
        my_rows = pl.ds(s * rows_per_tile, rows_per_tile)
        base = jnp.where(c == 0, s * bpt0,
                         _NUM_SUBCORES * bpt0 + s * bpt1)
        pltpu.sync_copy(z_hbm.at[my_rows], acc_sh.at[my_rows])
        plsc.subcore_barrier()

        def body(i):
            pltpu.sync_copy(sdb_hbm.at[pl.ds(base + i * nb, nb)], idx)
            descs = [
                pltpu.async_copy(m_hbm.at[idx.at[b].at[0]], rows.at[b],
                                 gsem[b])
                for b in range(nb)
            ]
            for b in range(nb):
                descs[b].wait()
                pltpu.sync_copy(rows.at[b], acc_sh.at[idx.at[b].at[1]],
                                add=True)

        @pl.loop(0, bpt1 // nb)
        def _(i):
            body(i)

        @pl.when(c == 0)
        def _():
            @pl.loop(bpt1 // nb, bpt0 // nb)
            def _(i):
                body(i)

        plsc.subcore_barrier()
        pltpu.sync_copy(acc_sh.at[my_rows], out_hbm.at[c].at[my_rows])

    return k(m, sdb, zeros)


def kernel(x, edge_index, W1_rel, W1_root, b1, W2_rel, W2_root, b2):
    N, D = x.shape
    E = edge_index.shape[1]
    nw = _NUM_CORES * _NUM_SUBCORES

    blocks_per_tile = _round_up(-(-E // (nw * _BLK_EDGES)), 3)
    tot_bpt = 2 * blocks_per_tile
    bpt0 = min(_round_up(int(tot_bpt * 0.81), 3), tot_bpt - 3)
    bpt1 = tot_bpt - bpt0
    e_pad = _NUM_SUBCORES * tot_bpt * _BLK_EDGES
    rows_per_tile = _round_up(-(-(N + 1) // _NUM_SUBCORES), 8)
    n_acc = _NUM_SUBCORES * rows_per_tile

    src = edge_index[0].astype(jnp.int32)
    dst = edge_index[1].astype(jnp.int32)
    pad = e_pad - E
    nblocks = e_pad // _BLK_EDGES
    srcb = jnp.pad(src, (0, pad), constant_values=N).reshape(nblocks,
                                                             _BLK_EDGES)
    dstb = jnp.pad(dst, (0, pad), constant_values=N).reshape(nblocks,
                                                             _BLK_EDGES)
    sdb = jnp.stack([srcb, dstb], axis=1)
    xp = jnp.pad(x, ((0, n_acc - N), (0, 0)))
    zeros = jnp.zeros((n_acc, D), jnp.float32)
    b1r = b1.reshape(1, D)
    b2r = b2.reshape(1, D)

    blk_rows = 1024 if n_acc % 1024 == 0 else 64

    m1, r1 = _dense_two(xp, W1_rel, W1_root, b1r, blk_rows)
    parts1 = _sc_segsum(m1, sdb, zeros, n_acc, rows_per_tile, bpt0, bpt1)
    m2, r2 = _fused_dense_two(parts1, r1, W2_rel, W2_root, b2r, blk_rows)
    parts2 = _sc_segsum(m2, sdb, zeros, n_acc, rows_per_tile, bpt0, bpt1)
    out = _combine(parts2, r2, blk_rows)
    return out[:N]


# speedup vs baseline: 2.0636x; 1.7302x over previous
import functools

import jax
import jax.numpy as jnp
from jax import lax
from jax.experimental import pallas as pl
from jax.experimental.pallas import tpu as pltpu
from jax.experimental.pallas import tpu_sc as plsc

_NUM_CORES = 2
_NUM_SUBCORES = 16
_BLK_EDGES = 128


def _round_up(a, m):
    return (a + m - 1) // m * m


def _dense_two(h, W_rel, W_root, b, blk_rows):
    R, D = h.shape

    def body(h_ref, wr_ref, wo_ref, b_ref, m_ref, r_ref):
        hb = h_ref[...]
        dn = (((1,), (1,)), ((), ()))
        m_ref[...] = lax.dot_general(hb, wr_ref[...], dn,
                                     preferred_element_type=jnp.float32)
        r_ref[...] = lax.dot_general(hb, wo_ref[...], dn,
                                     preferred_element_type=jnp.float32) + b_ref[...]

    return pl.pallas_call(
        body,
        grid=(R // blk_rows,),
        in_specs=[
            pl.BlockSpec((blk_rows, D), lambda i: (i, 0)),
            pl.BlockSpec((D, D), lambda i: (0, 0)),
            pl.BlockSpec((D, D), lambda i: (0, 0)),
            pl.BlockSpec((1, D), lambda i: (0, 0)),
        ],
        out_specs=[
            pl.BlockSpec((blk_rows, D), lambda i: (i, 0)),
            pl.BlockSpec((blk_rows, D), lambda i: (i, 0)),
        ],
        out_shape=[
            jax.ShapeDtypeStruct((R, D), jnp.float32),
            jax.ShapeDtypeStruct((R, D), jnp.float32),
        ],
    )(h, W_rel, W_root, b)


def _fused_dense_two(parts, r_prev, W_rel, W_root, b, blk_rows):
    _, R, D = parts.shape

    def body(p_ref, rp_ref, wr_ref, wo_ref, b_ref, m_ref, r_ref):
        hb = jnp.maximum(p_ref[0] + p_ref[1] + rp_ref[...], 0.0)
        dn = (((1,), (1,)), ((), ()))
        m_ref[...] = lax.dot_general(hb, wr_ref[...], dn,
                                     preferred_element_type=jnp.float32)
        r_ref[...] = lax.dot_general(hb, wo_ref[...], dn,
                                     preferred_element_type=jnp.float32) + b_ref[...]

    return pl.pallas_call(
        body,
        grid=(R // blk_rows,),
        in_specs=[
            pl.BlockSpec((2, blk_rows, D), lambda i: (0, i, 0)),
            pl.BlockSpec((blk_rows, D), lambda i: (i, 0)),
            pl.BlockSpec((D, D), lambda i: (0, 0)),
            pl.BlockSpec((D, D), lambda i: (0, 0)),
            pl.BlockSpec((1, D), lambda i: (0, 0)),
        ],
        out_specs=[
            pl.BlockSpec((blk_rows, D), lambda i: (i, 0)),
            pl.BlockSpec((blk_rows, D), lambda i: (i, 0)),
        ],
        out_shape=[
            jax.ShapeDtypeStruct((R, D), jnp.float32),
            jax.ShapeDtypeStruct((R, D), jnp.float32),
        ],
    )(parts, r_prev, W_rel, W_root, b)


def _combine(parts, r, blk_rows):
    _, R, D = parts.shape

    def body(p_ref, r_ref, o_ref):
        o_ref[...] = jnp.maximum(p_ref[0] + p_ref[1] + r_ref[...], 0.0)

    return pl.pallas_call(
        body,
        grid=(R // blk_rows,),
        in_specs=[
            pl.BlockSpec((2, blk_rows, D), lambda i: (0, i, 0)),
            pl.BlockSpec((blk_rows, D), lambda i: (i, 0)),
        ],
        out_specs=pl.BlockSpec((blk_rows, D), lambda i: (i, 0)),
        out_shape=jax.ShapeDtypeStruct((R, D), jnp.float32),
    )(parts, r)


def _sc_segsum(m, srcb, dstb, zeros, n_acc, rows_per_tile, blocks_per_tile):
    D = m.shape[1]
    mesh = plsc.VectorSubcoreMesh(core_axis_name="c", subcore_axis_name="s",
                                  num_cores=_NUM_CORES,
                                  num_subcores=_NUM_SUBCORES)

    @functools.partial(
        pl.kernel,
        out_type=jax.ShapeDtypeStruct((_NUM_CORES, n_acc, D), jnp.float32),
        mesh=mesh,
        scratch_types=[
            pltpu.VMEM((_BLK_EDGES,), jnp.int32),
            pltpu.VMEM((_BLK_EDGES,), jnp.int32),
            pltpu.VMEM((_BLK_EDGES, D), jnp.float32),
            pltpu.VMEM_SHARED((n_acc, D), jnp.float32),
            pltpu.SemaphoreType.DMA,
        ],
    )
    def k(m_hbm, srcb_hbm, dstb_hbm, z_hbm, out_hbm, idx_s, idx_d, rows_v,
          acc_sh, sem):
        c = lax.axis_index("c")
        s = lax.axis_index("s")
        my_rows = pl.ds(s * rows_per_tile, rows_per_tile)
        pltpu.sync_copy(z_hbm, acc_sh.at[my_rows])
        plsc.subcore_barrier()
        base = (c * _NUM_SUBCORES + s) * blocks_per_tile

        @pl.loop(0, blocks_per_tile)
        def _(j):
            blk = base + j
            pltpu.sync_copy(srcb_hbm.at[blk], idx_s)
            pltpu.sync_copy(dstb_hbm.at[blk], idx_d)
            pltpu.async_copy(m_hbm.at[idx_s], rows_v, sem).wait()
            pltpu.sync_copy(rows_v, acc_sh.at[idx_d], add=True)

        plsc.subcore_barrier()
        pltpu.sync_copy(acc_sh.at[my_rows], out_hbm.at[c].at[my_rows])

    return k(m, srcb, dstb, zeros)


def kernel(x, edge_index, W1_rel, W1_root, b1, W2_rel, W2_root, b2):
    N, D = x.shape
    E = edge_index.shape[1]
    nw = _NUM_CORES * _NUM_SUBCORES

    blocks_per_tile = -(-E // (nw * _BLK_EDGES))
    e_pad = nw * _BLK_EDGES * blocks_per_tile
    rows_per_tile = _round_up(-(-(N + 1) // _NUM_SUBCORES), 64)
    n_acc = _NUM_SUBCORES * rows_per_tile

    src = edge_index[0].astype(jnp.int32)
    dst = edge_index[1].astype(jnp.int32)
    pad = e_pad - E
    srcb = jnp.pad(src, (0, pad), constant_values=N).reshape(e_pad // _BLK_EDGES,
                                                             _BLK_EDGES)
    dstb = jnp.pad(dst, (0, pad), constant_values=N).reshape(e_pad // _BLK_EDGES,
                                                             _BLK_EDGES)
    xp = jnp.pad(x, ((0, n_acc - N), (0, 0)))
    zeros = jnp.zeros((rows_per_tile, D), jnp.float32)
    b1r = b1.reshape(1, D)
    b2r = b2.reshape(1, D)

    blk_rows = 1024 if n_acc % 1024 == 0 else 64

    m1, r1 = _dense_two(xp, W1_rel, W1_root, b1r, blk_rows)
    parts1 = _sc_segsum(m1, srcb, dstb, zeros, n_acc, rows_per_tile,
                        blocks_per_tile)
    m2, r2 = _fused_dense_two(parts1, r1, W2_rel, W2_root, b2r, blk_rows)
    parts2 = _sc_segsum(m2, srcb, dstb, zeros, n_acc, rows_per_tile,
                        blocks_per_tile)
    out = _combine(parts2, r2, blk_rows)
    return out[:N]


# speedup vs baseline: 2.2700x; 1.1000x over previous
import functools

import jax
import jax.numpy as jnp
from jax import lax
from jax.experimental import pallas as pl
from jax.experimental.pallas import tpu as pltpu
from jax.experimental.pallas import tpu_sc as plsc

_NUM_CORES = 2
_NUM_SUBCORES = 16
_BLK_EDGES = 128


def _round_up(a, m):
    return (a + m - 1) // m * m


def _dense_two(h, W_rel, W_root, b, blk_rows):
    R, D = h.shape

    def body(h_ref, wr_ref, wo_ref, b_ref, m_ref, r_ref):
        hb = h_ref[...]
        dn = (((1,), (1,)), ((), ()))
        m_ref[...] = lax.dot_general(hb, wr_ref[...], dn,
                                     preferred_element_type=jnp.float32)
        r_ref[...] = lax.dot_general(hb, wo_ref[...], dn,
                                     preferred_element_type=jnp.float32) + b_ref[...]

    return pl.pallas_call(
        body,
        grid=(R // blk_rows,),
        in_specs=[
            pl.BlockSpec((blk_rows, D), lambda i: (i, 0)),
            pl.BlockSpec((D, D), lambda i: (0, 0)),
            pl.BlockSpec((D, D), lambda i: (0, 0)),
            pl.BlockSpec((1, D), lambda i: (0, 0)),
        ],
        out_specs=[
            pl.BlockSpec((blk_rows, D), lambda i: (i, 0)),
            pl.BlockSpec((blk_rows, D), lambda i: (i, 0)),
        ],
        out_shape=[
            jax.ShapeDtypeStruct((R, D), jnp.float32),
            jax.ShapeDtypeStruct((R, D), jnp.float32),
        ],
    )(h, W_rel, W_root, b)


def _fused_dense_two(parts, r_prev, W_rel, W_root, b, blk_rows):
    _, R, D = parts.shape

    def body(p_ref, rp_ref, wr_ref, wo_ref, b_ref, m_ref, r_ref):
        hb = jnp.maximum(p_ref[0] + p_ref[1] + rp_ref[...], 0.0)
        dn = (((1,), (1,)), ((), ()))
        m_ref[...] = lax.dot_general(hb, wr_ref[...], dn,
                                     preferred_element_type=jnp.float32)
        r_ref[...] = lax.dot_general(hb, wo_ref[...], dn,
                                     preferred_element_type=jnp.float32) + b_ref[...]

    return pl.pallas_call(
        body,
        grid=(R // blk_rows,),
        in_specs=[
            pl.BlockSpec((2, blk_rows, D), lambda i: (0, i, 0)),
            pl.BlockSpec((blk_rows, D), lambda i: (i, 0)),
            pl.BlockSpec((D, D), lambda i: (0, 0)),
            pl.BlockSpec((D, D), lambda i: (0, 0)),
            pl.BlockSpec((1, D), lambda i: (0, 0)),
        ],
        out_specs=[
            pl.BlockSpec((blk_rows, D), lambda i: (i, 0)),
            pl.BlockSpec((blk_rows, D), lambda i: (i, 0)),
        ],
        out_shape=[
            jax.ShapeDtypeStruct((R, D), jnp.float32),
            jax.ShapeDtypeStruct((R, D), jnp.float32),
        ],
    )(parts, r_prev, W_rel, W_root, b)


def _combine(parts, r, blk_rows):
    _, R, D = parts.shape

    def body(p_ref, r_ref, o_ref):
        o_ref[...] = jnp.maximum(p_ref[0] + p_ref[1] + r_ref[...], 0.0)

    return pl.pallas_call(
        body,
        grid=(R // blk_rows,),
        in_specs=[
            pl.BlockSpec((2, blk_rows, D), lambda i: (0, i, 0)),
            pl.BlockSpec((blk_rows, D), lambda i: (i, 0)),
        ],
        out_specs=pl.BlockSpec((blk_rows, D), lambda i: (i, 0)),
        out_shape=jax.ShapeDtypeStruct((R, D), jnp.float32),
    )(parts, r)


def _sc_segsum(m, srcb, dstb, zeros, n_acc, rows_per_tile, bpt0, bpt1):
    D = m.shape[1]
    mesh = plsc.VectorSubcoreMesh(core_axis_name="c", subcore_axis_name="s",
                                  num_cores=_NUM_CORES,
                                  num_subcores=_NUM_SUBCORES)

    @functools.partial(
        pl.kernel,
        out_type=jax.ShapeDtypeStruct((_NUM_CORES, n_acc, D), jnp.float32),
        mesh=mesh,
        scratch_types=[
            pltpu.VMEM((_BLK_EDGES,), jnp.int32),
            pltpu.VMEM((_BLK_EDGES,), jnp.int32),
            pltpu.VMEM((_BLK_EDGES, D), jnp.float32),
            pltpu.VMEM_SHARED((n_acc, D), jnp.float32),
            pltpu.SemaphoreType.DMA,
        ],
    )
    def k(m_hbm, srcb_hbm, dstb_hbm, z_hbm, out_hbm, idx_s, idx_d, rows_v,
          acc_sh, sem):
        c = lax.axis_index("c")
        s = lax.axis_index("s")
        my_rows = pl.ds(s * rows_per_tile, rows_per_tile)
        pltpu.sync_copy(z_hbm, acc_sh.at[my_rows])
        plsc.subcore_barrier()
        base = jnp.where(c == 0, s * bpt0, _NUM_SUBCORES * bpt0 + s * bpt1)

        def body(j):
            blk = base + j
            pltpu.sync_copy(srcb_hbm.at[blk], idx_s)
            pltpu.sync_copy(dstb_hbm.at[blk], idx_d)
            pltpu.async_copy(m_hbm.at[idx_s], rows_v, sem).wait()
            pltpu.sync_copy(rows_v, acc_sh.at[idx_d], add=True)

        @pl.loop(0, bpt1)
        def _(j):
            body(j)

        @pl.when(c == 0)
        def _():
            @pl.loop(bpt1, bpt0)
            def _(j):
                body(j)

        plsc.subcore_barrier()
        pltpu.sync_copy(acc_sh.at[my_rows], out_hbm.at[c].at[my_rows])

    return k(m, srcb, dstb, zeros)


def kernel(x, edge_index, W1_rel, W1_root, b1, W2_rel, W2_root, b2):
    N, D = x.shape
    E = edge_index.shape[1]
    nw = _NUM_CORES * _NUM_SUBCORES

    blocks_per_tile = -(-E // (nw * _BLK_EDGES))
    tot_bpt = 2 * blocks_per_tile
    bpt0 = max(min(int(tot_bpt * 0.6), tot_bpt - 1), 1)
    bpt1 = tot_bpt - bpt0
    e_pad = nw * _BLK_EDGES * blocks_per_tile
    rows_per_tile = _round_up(-(-(N + 1) // _NUM_SUBCORES), 64)
    n_acc = _NUM_SUBCORES * rows_per_tile

    src = edge_index[0].astype(jnp.int32)
    dst = edge_index[1].astype(jnp.int32)
    pad = e_pad - E
    srcb = jnp.pad(src, (0, pad), constant_values=N).reshape(e_pad // _BLK_EDGES,
                                                             _BLK_EDGES)
    dstb = jnp.pad(dst, (0, pad), constant_values=N).reshape(e_pad // _BLK_EDGES,
                                                             _BLK_EDGES)
    xp = jnp.pad(x, ((0, n_acc - N), (0, 0)))
    zeros = jnp.zeros((rows_per_tile, D), jnp.float32)
    b1r = b1.reshape(1, D)
    b2r = b2.reshape(1, D)

    blk_rows = 1024 if n_acc % 1024 == 0 else 64

    m1, r1 = _dense_two(xp, W1_rel, W1_root, b1r, blk_rows)
    parts1 = _sc_segsum(m1, srcb, dstb, zeros, n_acc, rows_per_tile,
                        bpt0, bpt1)
    m2, r2 = _fused_dense_two(parts1, r1, W2_rel, W2_root, b2r, blk_rows)
    parts2 = _sc_segsum(m2, srcb, dstb, zeros, n_acc, rows_per_tile,
                        bpt0, bpt1)
    out = _combine(parts2, r2, blk_rows)
    return out[:N]
